# baseline (device time: 41856 ns/iter reference)
import jax
import jax.numpy as jnp
from jax import lax
from jax.experimental import pallas as pl
from jax.experimental.pallas import tpu as pltpu


def kernel(x, A, B, C):
    Bb, S, D = x.shape
    N = A.shape[1]

    dAT = jnp.exp(A).T

    CHUNK = 8
    n_chunks = S // CHUNK

    def body(x_ref, dAT_ref, B_ref, C_ref, out_ref,
             h_ref, hin_ref, send_sem, recv_sem):
        my_x = lax.axis_index("x")
        my_y = lax.axis_index("y")

        barrier = pltpu.get_barrier_semaphore()
        pl.semaphore_signal(
            barrier, inc=1,
            device_id=(my_x, 1 - my_y),
            device_id_type=pl.DeviceIdType.MESH,
        )
        pl.semaphore_wait(barrier, 1)

        copy = pltpu.make_async_remote_copy(
            src_ref=h_ref,
            dst_ref=hin_ref,
            send_sem=send_sem,
            recv_sem=recv_sem,
            device_id=(my_x, 1 - my_y),
            device_id_type=pl.DeviceIdType.MESH,
        )

        @pl.when(my_y == 0)
        def _():
            h_ref[...] = jnp.zeros_like(h_ref)

        @pl.when(my_y == 1)
        def _():
            copy.wait_recv()
            h_ref[...] = hin_ref[...]

        dAT_v = dAT_ref[...]

        def chunk_step(c, carry):
            t0 = c * CHUNK
            xc = x_ref[:, pl.ds(t0, CHUNK), :]
            Bc = B_ref[:, pl.ds(t0, CHUNK), :]
            Cc = C_ref[:, pl.ds(t0, CHUNK), :]
            yacc = [jnp.zeros((Bb, D), jnp.float32) for _ in range(CHUNK)]
            for n in range(N):
                h_n = h_ref[n]
                dA_n = dAT_v[n:n + 1, :]
                for j in range(CHUNK):
                    h_n = h_n * dA_n + xc[:, j, :] * Bc[:, j, n:n + 1]
                    yacc[j] = yacc[j] + h_n * Cc[:, j, n:n + 1]
                h_ref[n] = h_n
            out_ref[:, pl.ds(t0, CHUNK), :] = jnp.stack(yacc, axis=1)
            return carry

        lax.fori_loop(0, n_chunks, chunk_step, 0)

        @pl.when(my_y == 0)
        def _():
            copy.start()
            copy.wait_send()

    return pl.pallas_call(
        body,
        out_shape=jax.ShapeDtypeStruct((Bb, S, D), jnp.float32),
        in_specs=[pl.BlockSpec(memory_space=pltpu.VMEM)] * 4,
        out_specs=pl.BlockSpec(memory_space=pltpu.VMEM),
        scratch_shapes=[
            pltpu.VMEM((N, Bb, D), jnp.float32),
            pltpu.VMEM((N, Bb, D), jnp.float32),
            pltpu.SemaphoreType.DMA,
            pltpu.SemaphoreType.DMA,
        ],
        compiler_params=pltpu.CompilerParams(collective_id=0),
    )(x, dAT, B, C)


# device time: 23945 ns/iter; 1.7480x vs baseline; 1.7480x over previous
import jax
import jax.numpy as jnp
from jax import lax
from jax.experimental import pallas as pl
from jax.experimental.pallas import tpu as pltpu


def kernel(x, A, B, C):
    Bb, S, D = x.shape
    N = A.shape[1]
    P = N // 2
    CHUNK = 8
    n_chunks = S // CHUNK

    dAe = jnp.exp(A).T
    dA2 = jnp.repeat(dAe, Bb, axis=0).reshape(P, 2 * Bb, D)

    def pack(M):
        return (
            M.transpose(1, 2, 0)
            .reshape(S, P, 2, Bb)
            .transpose(0, 2, 3, 1)
            .reshape(S, 2 * Bb, P)
        )

    W = pack(B)
    V = pack(C)

    def body(x_ref, dA2_ref, W_ref, V_ref, out_ref,
             h_ref, hin_ref, send_sem, recv_sem):
        my_x = lax.axis_index("x")
        my_y = lax.axis_index("y")

        barrier = pltpu.get_barrier_semaphore()
        pl.semaphore_signal(
            barrier, inc=1,
            device_id=(my_x, 1 - my_y),
            device_id_type=pl.DeviceIdType.MESH,
        )
        pl.semaphore_wait(barrier, 1)

        copy = pltpu.make_async_remote_copy(
            src_ref=h_ref,
            dst_ref=hin_ref,
            send_sem=send_sem,
            recv_sem=recv_sem,
            device_id=(my_x, 1 - my_y),
            device_id_type=pl.DeviceIdType.MESH,
        )

        @pl.when(my_y == 0)
        def _():
            h_ref[...] = jnp.zeros_like(h_ref)

        @pl.when(my_y == 1)
        def _():
            copy.wait_recv()
            h_ref[...] = hin_ref[...]

        def chunk_step(c, carry):
            t0 = c * CHUNK
            xc = x_ref[:, pl.ds(t0, CHUNK), :]
            Wc = W_ref[pl.ds(t0, CHUNK)]
            Vc = V_ref[pl.ds(t0, CHUNK)]
            xj = [
                jnp.concatenate([xc[:, j, :], xc[:, j, :]], axis=0)
                for j in range(CHUNK)
            ]
            yacc = [jnp.zeros((2 * Bb, D), jnp.float32) for _ in range(CHUNK)]
            for p in range(P):
                h_p = h_ref[p]
                dA_p = dA2_ref[p]
                for j in range(CHUNK):
                    h_p = h_p * dA_p + xj[j] * Wc[j, :, p:p + 1]
                    yacc[j] = yacc[j] + h_p * Vc[j, :, p:p + 1]
                h_ref[p] = h_p
            yout = jnp.stack(
                [yacc[j][:Bb, :] + yacc[j][Bb:, :] for j in range(CHUNK)],
                axis=1,
            )
            out_ref[:, pl.ds(t0, CHUNK), :] = yout
            return carry

        lax.fori_loop(0, n_chunks, chunk_step, 0)

        @pl.when(my_y == 0)
        def _():
            copy.start()
            copy.wait_send()

    return pl.pallas_call(
        body,
        out_shape=jax.ShapeDtypeStruct((Bb, S, D), jnp.float32),
        in_specs=[pl.BlockSpec(memory_space=pltpu.VMEM)] * 4,
        out_specs=pl.BlockSpec(memory_space=pltpu.VMEM),
        scratch_shapes=[
            pltpu.VMEM((P, 2 * Bb, D), jnp.float32),
            pltpu.VMEM((P, 2 * Bb, D), jnp.float32),
            pltpu.SemaphoreType.DMA,
            pltpu.SemaphoreType.DMA,
        ],
        compiler_params=pltpu.CompilerParams(collective_id=0),
    )(x, dA2, W, V)


# device time: 22694 ns/iter; 1.8444x vs baseline; 1.0551x over previous
import jax
import jax.numpy as jnp
from jax import lax
from jax.experimental import pallas as pl
from jax.experimental.pallas import tpu as pltpu


def kernel(x, A, B, C):
    Bb, S, D = x.shape
    N = A.shape[1]
    P = N // 2
    CHUNK = 8
    n_chunks = S // CHUNK

    dAe = jnp.exp(A).T
    dA2 = jnp.repeat(dAe, Bb, axis=0).reshape(P, 2 * Bb, D)

    def pack(M):
        return (
            M.transpose(1, 2, 0)
            .reshape(S, P, 2, Bb)
            .transpose(0, 2, 3, 1)
            .reshape(S, 2 * Bb, P)
        )

    W = pack(B)
    V = pack(C)

    cdt = jnp.bfloat16
    x = x.astype(cdt)
    dA2 = dA2.astype(cdt)
    W = W.astype(cdt)
    V = V.astype(cdt)

    def body(x_ref, dA2_ref, W_ref, V_ref, out_ref,
             h_ref, hin_ref, send_sem, recv_sem):
        my_x = lax.axis_index("x")
        my_y = lax.axis_index("y")

        barrier = pltpu.get_barrier_semaphore()
        pl.semaphore_signal(
            barrier, inc=1,
            device_id=(my_x, 1 - my_y),
            device_id_type=pl.DeviceIdType.MESH,
        )
        pl.semaphore_wait(barrier, 1)

        copy = pltpu.make_async_remote_copy(
            src_ref=h_ref,
            dst_ref=hin_ref,
            send_sem=send_sem,
            recv_sem=recv_sem,
            device_id=(my_x, 1 - my_y),
            device_id_type=pl.DeviceIdType.MESH,
        )

        @pl.when(my_y == 0)
        def _():
            h_ref[...] = jnp.zeros_like(h_ref)

        @pl.when(my_y == 1)
        def _():
            copy.wait_recv()
            h_ref[...] = hin_ref[...]

        def chunk_step(c, carry):
            t0 = c * CHUNK
            xc = x_ref[:, pl.ds(t0, CHUNK), :]
            Wc = W_ref[pl.ds(t0, CHUNK)]
            Vc = V_ref[pl.ds(t0, CHUNK)]
            xj = [
                jnp.concatenate([xc[:, j, :], xc[:, j, :]], axis=0)
                for j in range(CHUNK)
            ]
            yacc = [jnp.zeros((2 * Bb, D), cdt) for _ in range(CHUNK)]
            for p in range(P):
                h_p = h_ref[p]
                dA_p = dA2_ref[p]
                for j in range(CHUNK):
                    h_p = h_p * dA_p + xj[j] * Wc[j, :, p:p + 1]
                    yacc[j] = yacc[j] + h_p * Vc[j, :, p:p + 1]
                h_ref[p] = h_p
            yout = jnp.stack(
                [yacc[j][:Bb, :] + yacc[j][Bb:, :] for j in range(CHUNK)],
                axis=1,
            )
            out_ref[:, pl.ds(t0, CHUNK), :] = yout.astype(jnp.float32)
            return carry

        lax.fori_loop(0, n_chunks, chunk_step, 0)

        @pl.when(my_y == 0)
        def _():
            copy.start()
            copy.wait_send()

    return pl.pallas_call(
        body,
        out_shape=jax.ShapeDtypeStruct((Bb, S, D), jnp.float32),
        in_specs=[pl.BlockSpec(memory_space=pltpu.VMEM)] * 4,
        out_specs=pl.BlockSpec(memory_space=pltpu.VMEM),
        scratch_shapes=[
            pltpu.VMEM((P, 2 * Bb, D), cdt),
            pltpu.VMEM((P, 2 * Bb, D), cdt),
            pltpu.SemaphoreType.DMA,
            pltpu.SemaphoreType.DMA,
        ],
        compiler_params=pltpu.CompilerParams(collective_id=0),
    )(x, dA2, W, V)


# device time: 20172 ns/iter; 2.0750x vs baseline; 1.1250x over previous
import jax
import jax.numpy as jnp
from jax import lax
from jax.experimental import pallas as pl
from jax.experimental.pallas import tpu as pltpu


def kernel(x, A, B, C):
    Bb, S, D = x.shape
    N = A.shape[1]
    P = N // 2
    CHUNK = 8
    n_chunks = S // CHUNK

    dAe = jnp.exp(A).T
    dA2 = jnp.repeat(dAe, Bb, axis=0).reshape(P, 2 * Bb, D)

    def pack(M):
        return (
            M.transpose(1, 2, 0)
            .reshape(S, P, 2, Bb)
            .transpose(0, 2, 3, 1)
            .reshape(S, 2 * Bb, P)
        )

    W = pack(B)
    V = pack(C)

    cdt = jnp.bfloat16
    x = x.astype(cdt)
    dA2 = dA2.astype(cdt)
    W = W.astype(cdt)
    V = V.astype(cdt)

    def body(x_ref, dA2_ref, W_ref, V_ref, out_ref,
             h_ref, hin_ref, send_sem, recv_sem):
        my_x = lax.axis_index("x")
        my_y = lax.axis_index("y")

        barrier = pltpu.get_barrier_semaphore()
        pl.semaphore_signal(
            barrier, inc=1,
            device_id=(my_x, 1 - my_y),
            device_id_type=pl.DeviceIdType.MESH,
        )
        pl.semaphore_wait(barrier, 1)

        copy = pltpu.make_async_remote_copy(
            src_ref=h_ref,
            dst_ref=hin_ref,
            send_sem=send_sem,
            recv_sem=recv_sem,
            device_id=(my_x, 1 - my_y),
            device_id_type=pl.DeviceIdType.MESH,
        )

        h_ref[...] = jnp.zeros_like(h_ref)

        def chunk_step(c, carry):
            t0 = c * CHUNK
            xc = x_ref[:, pl.ds(t0, CHUNK), :]
            Wc = W_ref[pl.ds(t0, CHUNK)]
            Vc = V_ref[pl.ds(t0, CHUNK)]
            xj = [
                jnp.concatenate([xc[:, j, :], xc[:, j, :]], axis=0)
                for j in range(CHUNK)
            ]
            yacc = [jnp.zeros((2 * Bb, D), cdt) for _ in range(CHUNK)]
            for p in range(P):
                h_p = h_ref[p]
                dA_p = dA2_ref[p]
                for j in range(CHUNK):
                    h_p = h_p * dA_p + xj[j] * Wc[j, :, p:p + 1]
                    yacc[j] = yacc[j] + h_p * Vc[j, :, p:p + 1]
                h_ref[p] = h_p
            yout = jnp.stack(
                [yacc[j][:Bb, :] + yacc[j][Bb:, :] for j in range(CHUNK)],
                axis=1,
            )
            out_ref[:, pl.ds(t0, CHUNK), :] = yout.astype(jnp.float32)
            return carry

        lax.fori_loop(0, n_chunks, chunk_step, 0)

        @pl.when(my_y == 0)
        def _():
            copy.start()
            copy.wait_send()

        @pl.when(my_y == 1)
        def _():
            copy.wait_recv()

            def corr_chunk(c, carry):
                t0 = c * CHUNK
                Vc = V_ref[pl.ds(t0, CHUNK)]
                cur = out_ref[:, pl.ds(t0, CHUNK), :]
                yacc = [jnp.zeros((2 * Bb, D), cdt) for _ in range(CHUNK)]
                for p in range(P):
                    h_p = hin_ref[p]
                    dA_p = dA2_ref[p]
                    for j in range(CHUNK):
                        h_p = h_p * dA_p
                        yacc[j] = yacc[j] + h_p * Vc[j, :, p:p + 1]
                    hin_ref[p] = h_p
                ycorr = jnp.stack(
                    [yacc[j][:Bb, :] + yacc[j][Bb:, :] for j in range(CHUNK)],
                    axis=1,
                )
                out_ref[:, pl.ds(t0, CHUNK), :] = cur + ycorr.astype(jnp.float32)
                return carry

            lax.fori_loop(0, n_chunks, corr_chunk, 0)

    return pl.pallas_call(
        body,
        out_shape=jax.ShapeDtypeStruct((Bb, S, D), jnp.float32),
        in_specs=[pl.BlockSpec(memory_space=pltpu.VMEM)] * 4,
        out_specs=pl.BlockSpec(memory_space=pltpu.VMEM),
        scratch_shapes=[
            pltpu.VMEM((P, 2 * Bb, D), cdt),
            pltpu.VMEM((P, 2 * Bb, D), cdt),
            pltpu.SemaphoreType.DMA,
            pltpu.SemaphoreType.DMA,
        ],
        compiler_params=pltpu.CompilerParams(collective_id=0),
    )(x, dA2, W, V)
